# Initial kernel scaffold; baseline (speedup 1.0000x reference)
import jax
import jax.numpy as jnp
from jax.experimental import pallas as pl


def kernel(bev_masks, scores, labels):
    raise NotImplementedError("write your pallas kernel here")


# speedup vs baseline: 50.8523x; 50.8523x over previous
import jax
import jax.numpy as jnp
from jax.experimental import pallas as pl
from jax.experimental.pallas import tpu as pltpu

BLK = 512
MP = 2560


def _nms_cell(labA_ref, labB_ref, scoB_ref, a_ref, b_ref,
              keep_out, ks_out, keep_scr, sup_ref, s_ref, kl_ref):
    bi = pl.program_id(0)
    bj = pl.program_id(1)

    @pl.when(bj == 0)
    def _init():
        sup_ref[...] = jnp.zeros_like(sup_ref)

    @pl.when(bj <= bi)
    def _work():
        A = a_ref[...]
        B = b_ref[...]
        inter = jax.lax.dot_general(
            A, B, dimension_numbers=(((1,), (1,)), ((), ())),
            preferred_element_type=jnp.float32)
        areaA = jnp.sum(A.astype(jnp.float32), axis=1)
        areaB = jnp.sum(B.astype(jnp.float32), axis=1)
        labA = labA_ref[0, 0, :]
        labB = labB_ref[0, 0, :]
        cond = (3.0 * inter > areaA[:, None] + areaB[None, :]) \
            & (labA[:, None] == labB[None, :])
        condf = cond.astype(jnp.float32)

        @pl.when(bj < bi)
        def _offdiag():
            kb = keep_scr[pl.ds(bj, 1), :]
            contrib = jax.lax.dot_general(
                kb, condf, dimension_numbers=(((1,), (0,)), ((), ())),
                preferred_element_type=jnp.float32)
            sup_ref[...] = sup_ref[...] + contrib

        @pl.when(bj == bi)
        def _diag():
            ii = jax.lax.broadcasted_iota(jnp.int32, (BLK, BLK), 0)
            jj = jax.lax.broadcasted_iota(jnp.int32, (BLK, BLK), 1)
            Sf = condf * (jj > ii).astype(jnp.float32)
            s_ref[...] = Sf
            kl_ref[...] = (sup_ref[...] == 0.0).astype(jnp.float32)

            @pl.when(jnp.max(Sf) > 0.0)
            def _serial():
                lane = jax.lax.broadcasted_iota(jnp.int32, (1, BLK), 1)

                def body(i, carry):
                    kl = kl_ref[...]
                    ki = jnp.sum(jnp.where(lane == i, kl, 0.0))
                    srow = s_ref[pl.ds(i, 1), :]
                    kl_ref[...] = kl * (1.0 - srow * ki)
                    return carry

                jax.lax.fori_loop(0, BLK, body, 0)

            kl = kl_ref[...]
            keep_scr[pl.ds(bi, 1), :] = kl
            keep_out[0, ...] = kl
            ks_out[0, ...] = kl * scoB_ref[0, ...]


def _run_nms(msp, labp, scop, nb):
    grid = (nb, nb)
    out_shape = [
        jax.ShapeDtypeStruct((nb, 1, BLK), jnp.float32),
        jax.ShapeDtypeStruct((nb, 1, BLK), jnp.float32),
    ]
    keep_f, ks = pl.pallas_call(
        _nms_cell,
        grid=grid,
        in_specs=[
            pl.BlockSpec((1, 1, BLK), lambda i, j: (jnp.minimum(i, j), 0, 0)),
            pl.BlockSpec((1, 1, BLK), lambda i, j: (i, 0, 0)),
            pl.BlockSpec((1, 1, BLK), lambda i, j: (i, 0, 0)),
            pl.BlockSpec((BLK, MP), lambda i, j: (jnp.minimum(i, j), 0)),
            pl.BlockSpec((BLK, MP), lambda i, j: (i, 0)),
        ],
        out_specs=[
            pl.BlockSpec((1, 1, BLK), lambda i, j: (i, 0, 0)),
            pl.BlockSpec((1, 1, BLK), lambda i, j: (i, 0, 0)),
        ],
        out_shape=out_shape,
        scratch_shapes=[
            pltpu.VMEM((nb, BLK), jnp.float32),
            pltpu.VMEM((1, BLK), jnp.float32),
            pltpu.VMEM((BLK, BLK), jnp.float32),
            pltpu.VMEM((1, BLK), jnp.float32),
        ],
        compiler_params=pltpu.CompilerParams(
            dimension_semantics=("arbitrary", "arbitrary")),
    )(labp, labp, scop, msp, msp)
    return keep_f, ks


def kernel(bev_masks, scores, labels):
    n, m = bev_masks.shape
    nb = (n + BLK - 1) // BLK
    npad = nb * BLK

    order = jnp.argsort(-scores)
    ms = bev_masks[order]
    labs = labels[order]
    scos = scores[order]

    msp = jnp.zeros((npad, MP), jnp.bfloat16)
    msp = msp.at[:n, :m].set(ms.astype(jnp.bfloat16))
    labp = jnp.full((npad,), -1, jnp.int32).at[:n].set(
        labs.astype(jnp.int32)).reshape(nb, 1, BLK)
    scop = jnp.zeros((npad,), jnp.float32).at[:n].set(scos).reshape(nb, 1, BLK)

    keep_f, ks = _run_nms(msp, labp, scop, nb)

    keep = keep_f.reshape(npad)[:n] > 0.5
    kept_scores = ks.reshape(npad)[:n]
    return order, keep, kept_scores
